# baseline (speedup 1.0000x reference)
import jax
import jax.numpy as jnp
import numpy as np
from jax import lax
from jax.experimental import pallas as pl
from jax.experimental.pallas import tpu as pltpu
from jax.experimental.pallas import tpu_sc as plsc

_TOKENS = 32768
_D = 768
_E = 64
_NPH = 4
_TOK_P = _TOKENS // _NPH
_TC_TILE = 4096
_NW = 32
_TPW = _TOK_P // _NW
_NGRP = _TPW // 16

_NEG_INF = np.float32(-np.inf)


def _logits_body(x_ref, w_ref, b_ref, o_ref):
    w = w_ref[...]
    bias = b_ref[...]
    lt = lax.dot_general(
        w, x_ref[...], (((1,), (1,)), ((), ())),
        preferred_element_type=jnp.float32) + bias
    for j in range(_TC_TILE // _TPW):
        o_ref[j] = lt[:, j * _TPW:(j + 1) * _TPW]


def _argcmp_merge(mv, mi, cv, ci):
    take = (cv > mv) | ((cv == mv) & (ci < mi))
    return jnp.where(take, cv, mv), jnp.where(take, ci, mi)


def _route_body(lg_hbm, ow_hbm, oe_hbm, in_buf, ow_buf, oe_buf):
    cid = lax.axis_index("c")
    sid = lax.axis_index("s")
    wid = sid * 2 + cid
    pltpu.sync_copy(lg_hbm.at[wid], in_buf)

    def gbody(g, _):
        tok = g * 16
        m1a = in_buf[0, pl.ds(tok, 16)]
        m1b = in_buf[1, pl.ds(tok, 16)]
        i1a = jnp.zeros((16,), jnp.int32)
        i1b = jnp.ones((16,), jnp.int32)
        m2a = jnp.full((16,), _NEG_INF)
        m2b = jnp.full((16,), _NEG_INF)
        i2a = jnp.zeros((16,), jnp.int32)
        i2b = jnp.zeros((16,), jnp.int32)
        for e in range(2, _E, 2):
            va = in_buf[e, pl.ds(tok, 16)]
            vb = in_buf[e + 1, pl.ds(tok, 16)]
            ea = jnp.full((16,), np.int32(e))
            eb = jnp.full((16,), np.int32(e + 1))
            ca1 = va > m1a
            cb1 = vb > m1b
            ca2 = va > m2a
            cb2 = vb > m2b
            m2a = jnp.where(ca1, m1a, jnp.where(ca2, va, m2a))
            i2a = jnp.where(ca1, i1a, jnp.where(ca2, ea, i2a))
            m2b = jnp.where(cb1, m1b, jnp.where(cb2, vb, m2b))
            i2b = jnp.where(cb1, i1b, jnp.where(cb2, eb, i2b))
            m1a = jnp.where(ca1, va, m1a)
            i1a = jnp.where(ca1, ea, i1a)
            m1b = jnp.where(cb1, vb, m1b)
            i1b = jnp.where(cb1, eb, i1b)
        takeb = (m1b > m1a) | ((m1b == m1a) & (i1b < i1a))
        v1 = jnp.where(takeb, m1b, m1a)
        e1 = jnp.where(takeb, i1b, i1a)
        lv = jnp.where(takeb, m1a, m1b)
        li = jnp.where(takeb, i1a, i1b)
        v2, e2 = _argcmp_merge(lv, li, m2a, i2a)
        v2, e2 = _argcmp_merge(v2, e2, m2b, i2b)
        s = jnp.exp(v2 - v1)
        w2 = s / (1.0 + s)
        ow_buf[pl.ds(tok, 16)] = w2
        oe_buf[pl.ds(tok, 16)] = e1 * np.int32(64) + e2
        return 0

    lax.fori_loop(0, _NGRP, gbody, 0)
    base = wid * _TPW
    pltpu.sync_copy(ow_buf, ow_hbm.at[pl.ds(base, _TPW)])
    pltpu.sync_copy(oe_buf, oe_hbm.at[pl.ds(base, _TPW)])


def _expand_body(w2_ref, ep_ref, o_ref):
    w2t = jnp.transpose(w2_ref[...])
    ept = jnp.transpose(ep_ref[...])
    col = lax.broadcasted_iota(jnp.int32, (128, _E), 1)
    for s in range(_TC_TILE // 128):
        ep = ept[:, s:s + 1]
        w2 = w2t[:, s:s + 1]
        e1 = ep >> 6
        e2 = ep & np.int32(63)
        w1 = 1.0 - w2
        o_ref[pl.ds(s * 128, 128), :] = (
            jnp.where(col == e1, w1, 0.0) + jnp.where(col == e2, w2, 0.0))


def kernel(payload_tensor, W, b):
    b2 = b.reshape(_E, 1)

    route = pl.kernel(
        _route_body,
        out_type=(jax.ShapeDtypeStruct((_TOK_P,), jnp.float32),
                  jax.ShapeDtypeStruct((_TOK_P,), jnp.int32)),
        mesh=plsc.VectorSubcoreMesh(core_axis_name="c", subcore_axis_name="s"),
        compiler_params=pltpu.CompilerParams(needs_layout_passes=False),
        scratch_types=[
            pltpu.VMEM((_E, _TPW), jnp.float32),
            pltpu.VMEM((_TPW,), jnp.float32),
            pltpu.VMEM((_TPW,), jnp.int32),
        ],
    )

    w2s, eps = [], []
    for p in range(_NPH):
        logits_p = pl.pallas_call(
            _logits_body,
            grid=(_TOK_P // _TC_TILE,),
            in_specs=[
                pl.BlockSpec((_TC_TILE, _D), lambda i, p=p: (p * 2 + i, 0)),
                pl.BlockSpec((_E, _D), lambda i: (0, 0)),
                pl.BlockSpec((_E, 1), lambda i: (0, 0)),
            ],
            out_specs=pl.BlockSpec(
                (_TC_TILE // _TPW, _E, _TPW), lambda i: (i, 0, 0)),
            out_shape=jax.ShapeDtypeStruct((_NW, _E, _TPW), jnp.float32),
        )(payload_tensor, W, b2)
        w2_p, ep_p = route(logits_p)
        w2s.append(w2_p)
        eps.append(ep_p)

    w2r = jnp.concatenate(w2s).reshape(_TOKENS // 128, 128)
    epr = jnp.concatenate(eps).reshape(_TOKENS // 128, 128)

    return pl.pallas_call(
        _expand_body,
        grid=(_TOKENS // _TC_TILE,),
        in_specs=[
            pl.BlockSpec((_TC_TILE // 128, 128), lambda i: (i, 0)),
            pl.BlockSpec((_TC_TILE // 128, 128), lambda i: (i, 0)),
        ],
        out_specs=pl.BlockSpec((_TC_TILE, _E), lambda i: (i, 0)),
        out_shape=jax.ShapeDtypeStruct((_TOKENS, _E), jnp.float32),
    )(w2r, epr)


# speedup vs baseline: 1.0585x; 1.0585x over previous
import jax
import jax.numpy as jnp
import numpy as np
from jax import lax
from jax.experimental import pallas as pl
from jax.experimental.pallas import tpu as pltpu
from jax.experimental.pallas import tpu_sc as plsc

_TOKENS = 32768
_D = 768
_E = 64
_NPH = 4
_TOK_P = _TOKENS // _NPH
_TC_TILE = 2048
_NW = 32
_TPW = _TOK_P // _NW
_NGRP = _TPW // 16

_NEG_INF = np.float32(-np.inf)


def _logits_body(x_ref, w_ref, b_ref, o_ref):
    w = w_ref[...]
    bias = b_ref[...]
    lt = lax.dot_general(
        w, x_ref[...], (((1,), (1,)), ((), ())),
        preferred_element_type=jnp.float32) + bias
    for j in range(_TC_TILE // _TPW):
        o_ref[j] = lt[:, j * _TPW:(j + 1) * _TPW]


def _argcmp_merge(mv, mi, cv, ci):
    take = (cv > mv) | ((cv == mv) & (ci < mi))
    return jnp.where(take, cv, mv), jnp.where(take, ci, mi)


def _route_body(lg_hbm, out_hbm, in_buf, out_buf):
    cid = lax.axis_index("c")
    sid = lax.axis_index("s")
    wid = sid * 2 + cid
    lane = lax.iota(jnp.int32, 16)
    zero16 = jnp.zeros((16,), jnp.float32)

    def zbody(r, _):
        for c4 in range(_E // 16):
            out_buf[r, pl.ds(c4 * 16, 16)] = zero16
        return 0
    lax.fori_loop(0, _TPW, zbody, 0)

    pltpu.sync_copy(lg_hbm.at[wid], in_buf)

    def gbody(g, _):
        tok = g * 16
        m1a = in_buf[0, pl.ds(tok, 16)]
        m1b = in_buf[1, pl.ds(tok, 16)]
        i1a = jnp.zeros((16,), jnp.int32)
        i1b = jnp.ones((16,), jnp.int32)
        m2a = jnp.full((16,), _NEG_INF)
        m2b = jnp.full((16,), _NEG_INF)
        i2a = jnp.zeros((16,), jnp.int32)
        i2b = jnp.zeros((16,), jnp.int32)
        for e in range(2, _E, 2):
            va = in_buf[e, pl.ds(tok, 16)]
            vb = in_buf[e + 1, pl.ds(tok, 16)]
            ea = jnp.full((16,), np.int32(e))
            eb = jnp.full((16,), np.int32(e + 1))
            ca1 = va > m1a
            cb1 = vb > m1b
            ca2 = va > m2a
            cb2 = vb > m2b
            m2a = jnp.where(ca1, m1a, jnp.where(ca2, va, m2a))
            i2a = jnp.where(ca1, i1a, jnp.where(ca2, ea, i2a))
            m2b = jnp.where(cb1, m1b, jnp.where(cb2, vb, m2b))
            i2b = jnp.where(cb1, i1b, jnp.where(cb2, eb, i2b))
            m1a = jnp.where(ca1, va, m1a)
            i1a = jnp.where(ca1, ea, i1a)
            m1b = jnp.where(cb1, vb, m1b)
            i1b = jnp.where(cb1, eb, i1b)
        takeb = (m1b > m1a) | ((m1b == m1a) & (i1b < i1a))
        v1 = jnp.where(takeb, m1b, m1a)
        e1 = jnp.where(takeb, i1b, i1a)
        lv = jnp.where(takeb, m1a, m1b)
        li = jnp.where(takeb, i1a, i1b)
        v2, e2 = _argcmp_merge(lv, li, m2a, i2a)
        v2, e2 = _argcmp_merge(v2, e2, m2b, i2b)
        s = jnp.exp(v2 - v1)
        w2 = s / (1.0 + s)
        w1 = 1.0 - w2
        row = lane + g * 16
        plsc.store_scatter(out_buf, [row, e1], w1)
        plsc.store_scatter(out_buf, [row, e2], w2)
        return 0

    lax.fori_loop(0, _NGRP, gbody, 0)
    pltpu.sync_copy(out_buf, out_hbm.at[pl.ds(wid * _TPW, _TPW)])


def kernel(payload_tensor, W, b):
    b2 = b.reshape(_E, 1)

    route = pl.kernel(
        _route_body,
        out_type=jax.ShapeDtypeStruct((_TOK_P, _E), jnp.float32),
        mesh=plsc.VectorSubcoreMesh(core_axis_name="c", subcore_axis_name="s"),
        compiler_params=pltpu.CompilerParams(needs_layout_passes=False),
        scratch_types=[
            pltpu.VMEM((_E, _TPW), jnp.float32),
            pltpu.VMEM((_TPW, _E), jnp.float32),
        ],
    )

    tiles_per_phase = _TOK_P // _TC_TILE
    outs = []
    for p in range(_NPH):
        logits_p = pl.pallas_call(
            _logits_body,
            grid=(tiles_per_phase,),
            in_specs=[
                pl.BlockSpec(
                    (_TC_TILE, _D),
                    lambda i, p=p: (p * tiles_per_phase + i, 0)),
                pl.BlockSpec((_E, _D), lambda i: (0, 0)),
                pl.BlockSpec((_E, 1), lambda i: (0, 0)),
            ],
            out_specs=pl.BlockSpec(
                (_TC_TILE // _TPW, _E, _TPW), lambda i: (i, 0, 0)),
            out_shape=jax.ShapeDtypeStruct((_NW, _E, _TPW), jnp.float32),
        )(payload_tensor, W, b2)
        outs.append(route(logits_p))

    return jnp.concatenate(outs, axis=0)


# speedup vs baseline: 1.2726x; 1.2023x over previous
import jax
import jax.numpy as jnp
import numpy as np
from jax import lax
from jax.experimental import pallas as pl
from jax.experimental.pallas import tpu as pltpu
from jax.experimental.pallas import tpu_sc as plsc

_TOKENS = 32768
_D = 768
_E = 64
_TC_TILE = 4096
_NW = 32
_TPW = _TOKENS // _NW
_CH = 512
_NCH = _TPW // _CH
_NGRP = _CH // 16

_NEG_INF = np.float32(-np.inf)


def _logits_body(x_ref, w_ref, b_ref, o_ref):
    w = w_ref[...]
    bias = b_ref[...]
    for j in range(_TC_TILE // _TPW):
        x = x_ref[pl.ds(j * _TPW, _TPW), :]
        lt = lax.dot_general(
            w, x, (((1,), (1,)), ((), ())),
            preferred_element_type=jnp.float32) + bias
        for ci in range(_NCH):
            o_ref[j, ci] = lt[:, ci * _CH:(ci + 1) * _CH]


def _argcmp_merge(mv, mi, cv, ci):
    take = (cv > mv) | ((cv == mv) & (ci < mi))
    return jnp.where(take, cv, mv), jnp.where(take, ci, mi)


def _route_body(lg_hbm, out_hbm, in_buf, out_buf, stash, sem):
    cid = lax.axis_index("c")
    sid = lax.axis_index("s")
    wid = sid * 2 + cid
    lane = lax.iota(jnp.int32, 16)
    zero16 = jnp.zeros((16,), jnp.float32)

    first = pltpu.async_copy(lg_hbm.at[wid, 0], in_buf, sem)

    def zbody(r, _):
        for c4 in range(_E // 16):
            out_buf[r, pl.ds(c4 * 16, 16)] = zero16
        return 0
    lax.fori_loop(0, _CH, zbody, 0)

    for c in range(_NCH):
        if c == 0:
            first.wait()
        else:
            pltpu.sync_copy(lg_hbm.at[wid, c], in_buf)
        if c > 0:
            def rzbody(g, _):
                row = lane + g * 16
                plsc.store_scatter(out_buf, [row, stash[2 * g]], zero16)
                plsc.store_scatter(out_buf, [row, stash[2 * g + 1]], zero16)
                return 0
            lax.fori_loop(0, _NGRP, rzbody, 0)

        def gbody(g, _):
            tok = g * 16
            m1a = in_buf[0, pl.ds(tok, 16)]
            m1b = in_buf[1, pl.ds(tok, 16)]
            i1a = jnp.zeros((16,), jnp.int32)
            i1b = jnp.ones((16,), jnp.int32)
            m2a = jnp.full((16,), _NEG_INF)
            m2b = jnp.full((16,), _NEG_INF)
            i2a = jnp.zeros((16,), jnp.int32)
            i2b = jnp.zeros((16,), jnp.int32)
            for e in range(2, _E, 2):
                va = in_buf[e, pl.ds(tok, 16)]
                vb = in_buf[e + 1, pl.ds(tok, 16)]
                ea = jnp.full((16,), np.int32(e))
                eb = jnp.full((16,), np.int32(e + 1))
                ca1 = va > m1a
                cb1 = vb > m1b
                ca2 = va > m2a
                cb2 = vb > m2b
                m2a = jnp.where(ca1, m1a, jnp.where(ca2, va, m2a))
                i2a = jnp.where(ca1, i1a, jnp.where(ca2, ea, i2a))
                m2b = jnp.where(cb1, m1b, jnp.where(cb2, vb, m2b))
                i2b = jnp.where(cb1, i1b, jnp.where(cb2, eb, i2b))
                m1a = jnp.where(ca1, va, m1a)
                i1a = jnp.where(ca1, ea, i1a)
                m1b = jnp.where(cb1, vb, m1b)
                i1b = jnp.where(cb1, eb, i1b)
            takeb = (m1b > m1a) | ((m1b == m1a) & (i1b < i1a))
            v1 = jnp.where(takeb, m1b, m1a)
            e1 = jnp.where(takeb, i1b, i1a)
            lv = jnp.where(takeb, m1a, m1b)
            li = jnp.where(takeb, i1a, i1b)
            v2, e2 = _argcmp_merge(lv, li, m2a, i2a)
            v2, e2 = _argcmp_merge(v2, e2, m2b, i2b)
            s = jnp.exp(v2 - v1)
            w2 = s / (1.0 + s)
            w1 = 1.0 - w2
            row = lane + g * 16
            plsc.store_scatter(out_buf, [row, e1], w1)
            plsc.store_scatter(out_buf, [row, e2], w2)
            stash[2 * g] = e1
            stash[2 * g + 1] = e2
            return 0

        lax.fori_loop(0, _NGRP, gbody, 0)
        tok0 = wid * _TPW + c * _CH
        pltpu.sync_copy(out_buf, out_hbm.at[pl.ds(tok0, _CH)])


def kernel(payload_tensor, W, b):
    b2 = b.reshape(_E, 1)
    logits = pl.pallas_call(
        _logits_body,
        grid=(_TOKENS // _TC_TILE,),
        in_specs=[
            pl.BlockSpec((_TC_TILE, _D), lambda i: (i, 0)),
            pl.BlockSpec((_E, _D), lambda i: (0, 0)),
            pl.BlockSpec((_E, 1), lambda i: (0, 0)),
        ],
        out_specs=pl.BlockSpec(
            (_TC_TILE // _TPW, _NCH, _E, _CH), lambda i: (i, 0, 0, 0)),
        out_shape=jax.ShapeDtypeStruct((_NW, _NCH, _E, _CH), jnp.float32),
    )(payload_tensor, W, b2)

    route = pl.kernel(
        _route_body,
        out_type=jax.ShapeDtypeStruct((_TOKENS, _E), jnp.float32),
        mesh=plsc.VectorSubcoreMesh(core_axis_name="c", subcore_axis_name="s"),
        compiler_params=pltpu.CompilerParams(needs_layout_passes=False),
        scratch_types=[
            pltpu.VMEM((_E, _CH), jnp.float32),
            pltpu.VMEM((_CH, _E), jnp.float32),
            pltpu.VMEM((2 * _NGRP, 16), jnp.int32),
            pltpu.SemaphoreType.DMA,
        ],
    )
    return route(logits)
